# baseline (speedup 1.0000x reference)
import math

import jax
import jax.numpy as jnp
from jax.experimental import pallas as pl
from jax.experimental.pallas import tpu as pltpu

_H = 64
_L = 512
_B = 128
_SLOTS = 6
_V = 64
_DH = 32
_BB = 8
_NPROG = _B // _BB
_T = _BB * _L


def _ln(x, w, b):
    mu = jnp.mean(x, axis=1, keepdims=True)
    xc = x - mu
    var = jnp.mean(xc * xc, axis=1, keepdims=True)
    return xc / jnp.sqrt(var + 1e-5) * w + b


def _fused_kernel(
    seq_ref, query_ref, target_ref,
    ehi_ref, emid_ref, elo_ref,
    wq0_ref, wq1_ref, wk0_ref, wk1_ref, wv0_ref, wv1_ref,
    bq0_ref, bq1_ref, bk0_ref, bk1_ref, bv0_ref, bv1_ref,
    ao_ref, aob_ref,
    ff1w_ref, ff1b_ref, ff2w_ref, ff2b_ref,
    ln1w_ref, ln1b_ref, ln2w_ref, ln2b_ref,
    gatew_ref, gateb_ref,
    qemb_ref, qpw_ref, qpb_ref, routw_ref, routb_ref,
    out_ref,
    h_s, qs0, qs1, ks0, ks1, vs0, vs1, as0, as1, h2b_s, s_s, qr_s, tgt_s,
):
    f32 = jnp.float32
    bf16 = jnp.bfloat16
    inv_h = 1.0 / math.sqrt(float(_H))

    iota_bb = jax.lax.broadcasted_iota(jnp.int32, (_BB, _V), 1)
    qoh = (iota_bb == query_ref[:, :]).astype(f32)
    qh_e = jnp.dot(qoh, qemb_ref[:, :], preferred_element_type=f32)
    qr_s[:, :] = jnp.dot(qh_e, qpw_ref[:, :], preferred_element_type=f32) + qpb_ref[:, :]
    tgt_s[:, :] = (iota_bb == target_ref[:, :]).astype(f32)

    iota_tok = jax.lax.broadcasted_iota(jnp.int32, (_T, _V), 1)
    oh = (iota_tok == seq_ref[:, :]).astype(bf16)
    h = (jnp.dot(oh, ehi_ref[:, :], preferred_element_type=f32)
         + jnp.dot(oh, emid_ref[:, :], preferred_element_type=f32)
         ) + jnp.dot(oh, elo_ref[:, :], preferred_element_type=f32)
    h_s[:, :] = h
    qs0[:, :] = jnp.dot(h, wq0_ref[:, :], preferred_element_type=f32) + bq0_ref[:, :]
    qs1[:, :] = jnp.dot(h, wq1_ref[:, :], preferred_element_type=f32) + bq1_ref[:, :]
    ks0[:, :] = jnp.dot(h, wk0_ref[:, :], preferred_element_type=f32) + bk0_ref[:, :]
    ks1[:, :] = jnp.dot(h, wk1_ref[:, :], preferred_element_type=f32) + bk1_ref[:, :]
    vs0[:, :] = jnp.dot(h, wv0_ref[:, :], preferred_element_type=f32) + bv0_ref[:, :]
    vs1[:, :] = jnp.dot(h, wv1_ref[:, :], preferred_element_type=f32) + bv1_ref[:, :]

    scale = jnp.sqrt(jnp.float32(float(_DH)))

    def attn_unit(base, q_ref, k_ref, v_ref, a_ref):
        q = q_ref[base:base + _L, :]
        k = k_ref[base:base + _L, :]
        lg = jax.lax.dot_general(q, k, (((1,), (1,)), ((), ())),
                                 preferred_element_type=f32) / scale
        p = jnp.exp(lg - jnp.max(lg, axis=1, keepdims=True))
        att = p / jnp.sum(p, axis=1, keepdims=True)
        a_ref[base:base + _L, :] = jnp.dot(att, v_ref[base:base + _L, :],
                                           preferred_element_type=f32)

    for r in range(_BB):
        attn_unit(r * _L, qs0, ks0, vs0, as0)
        attn_unit(r * _L, qs1, ks1, vs1, as1)

    a_cat = jnp.concatenate([as0[:, :], as1[:, :]], axis=1)
    attn = jnp.dot(a_cat, ao_ref[:, :], preferred_element_type=f32) + aob_ref[:, :]
    h1 = _ln(h_s[:, :] + attn, ln1w_ref[:, :], ln1b_ref[:, :])
    ffa = jnp.maximum(
        jnp.dot(h1, ff1w_ref[:, :], preferred_element_type=f32) + ff1b_ref[:, :], 0.0)
    ff = jnp.dot(ffa, ff2w_ref[:, :], preferred_element_type=f32) + ff2b_ref[:, :]
    h2 = _ln(h1 + ff, ln2w_ref[:, :], ln2b_ref[:, :])
    h2b_s[:, :] = jnp.concatenate([h2.astype(bf16), jnp.ones((_T, 1), bf16)], axis=1)
    w_sel = jnp.concatenate(
        [gatew_ref[:, :], jnp.zeros((_H, 7), f32), jnp.transpose(qr_s[:, :])],
        axis=1)
    s_s[:, :] = jnp.dot(h2, w_sel, preferred_element_type=f32)

    st_all = jnp.transpose(s_s[:, :])
    row_iota = jax.lax.broadcasted_iota(jnp.int32, (_BB, _T), 0)
    lane_iota = jax.lax.broadcasted_iota(jnp.int32, (_BB, _T), 1)
    blockmask = (lane_iota // _L) == row_iota
    neg_inf = jnp.float32(-jnp.inf)

    g8 = jnp.where(blockmask,
                   jnp.broadcast_to(st_all[0:1, :], (_BB, _T)), neg_inf)
    qs8 = st_all[8:16, :] * inv_h

    cur = g8
    sel = jnp.zeros((_BB, _T), jnp.bool_)
    for _ in range(_SLOTS):
        m = jnp.max(cur, axis=1, keepdims=True)
        idx = jnp.min(jnp.where(cur == m, lane_iota, _T), axis=1, keepdims=True)
        hit = lane_iota == idx
        sel = jnp.logical_or(sel, hit)
        cur = jnp.where(hit, neg_inf, cur)

    qsm = jnp.where(sel, qs8, neg_inf)
    ms = jnp.max(qsm, axis=1, keepdims=True)
    e = jnp.exp(qsm - ms)
    pooled_e = jnp.dot(e.astype(bf16), h2b_s[:, :],
                       preferred_element_type=f32)
    pooled = pooled_e[:, 0:_H] * (1.0 / pooled_e[:, _H:_H + 1])
    logits = jnp.dot(pooled, routw_ref[:, :], preferred_element_type=f32) + routb_ref[:, :]
    mx = jnp.max(logits, axis=1, keepdims=True)
    lse = mx + jnp.log(jnp.sum(jnp.exp(logits - mx), axis=1, keepdims=True))
    lp = jnp.sum(tgt_s[:, :] * logits, axis=1, keepdims=True) - lse
    total = -jnp.sum(lp)

    out_ref[:, :, :] = jnp.full((1, 1, 128), total, f32)


def kernel(seq, query, target, embed_table, in_proj_w, in_proj_b, attn_out_w,
           attn_out_b, ff1_w, ff1_b, ff2_w, ff2_b, ln1_w, ln1_b, ln2_w, ln2_b,
           gate_w, gate_b, query_embed, qproj_w, qproj_b, rout_w, rout_b):
    f32 = jnp.float32
    bf16 = jnp.bfloat16
    seq2 = seq.reshape(_B * _L, 1).astype(jnp.int32)
    q2 = query.reshape(_B, 1).astype(jnp.int32)
    t2 = target.reshape(_B, 1).astype(jnp.int32)

    et = embed_table.astype(f32)
    ehi = et.astype(bf16)
    emid = (et - ehi.astype(f32)).astype(bf16)
    elo = (et - ehi.astype(f32) - emid.astype(f32)).astype(bf16)

    wq0 = in_proj_w[0:32].T
    wq1 = in_proj_w[32:64].T
    wk0 = in_proj_w[64:96].T
    wk1 = in_proj_w[96:128].T
    wv0 = in_proj_w[128:160].T
    wv1 = in_proj_w[160:192].T
    bq0 = in_proj_b[0:32].reshape(1, 32)
    bq1 = in_proj_b[32:64].reshape(1, 32)
    bk0 = in_proj_b[64:96].reshape(1, 32)
    bk1 = in_proj_b[96:128].reshape(1, 32)
    bv0 = in_proj_b[128:160].reshape(1, 32)
    bv1 = in_proj_b[160:192].reshape(1, 32)
    aoT = attn_out_w.T
    aob = attn_out_b.reshape(1, _H)
    ff1wT = ff1_w.T
    ff1b2 = ff1_b.reshape(1, 2 * _H)
    ff2wT = ff2_w.T
    ff2b2 = ff2_b.reshape(1, _H)
    ln1w2 = ln1_w.reshape(1, _H)
    ln1b2 = ln1_b.reshape(1, _H)
    ln2w2 = ln2_w.reshape(1, _H)
    ln2b2 = ln2_b.reshape(1, _H)
    gatew2 = gate_w.reshape(1, _H).T
    gateb2 = gate_b.reshape(1, 1)
    qpwT = qproj_w.T
    qpb2 = qproj_b.reshape(1, _H)
    routwT = rout_w.T
    routb2 = rout_b.reshape(1, _V)

    def full_spec(a):
        shp = a.shape
        return pl.BlockSpec(shp, lambda i, _n=len(shp): (0,) * _n)

    operands = [
        seq2, q2, t2,
        ehi, emid, elo,
        wq0, wq1, wk0, wk1, wv0, wv1,
        bq0, bq1, bk0, bk1, bv0, bv1,
        aoT, aob,
        ff1wT, ff1b2, ff2wT, ff2b2,
        ln1w2, ln1b2, ln2w2, ln2b2,
        gatew2, gateb2,
        query_embed, qpwT, qpb2, routwT, routb2,
    ]
    in_specs = [
        pl.BlockSpec((_T, 1), lambda i: (i, 0)),
        pl.BlockSpec((_BB, 1), lambda i: (i, 0)),
        pl.BlockSpec((_BB, 1), lambda i: (i, 0)),
    ] + [full_spec(a) for a in operands[3:]]

    partial = pl.pallas_call(
        _fused_kernel,
        grid=(_NPROG,),
        in_specs=in_specs,
        out_specs=pl.BlockSpec((1, 1, 128), lambda i: (i, 0, 0)),
        out_shape=jax.ShapeDtypeStruct((_NPROG, 1, 128), f32),
        scratch_shapes=[
            pltpu.VMEM((_T, _H), f32),
            pltpu.VMEM((_T, _DH), f32),
            pltpu.VMEM((_T, _DH), f32),
            pltpu.VMEM((_T, _DH), f32),
            pltpu.VMEM((_T, _DH), f32),
            pltpu.VMEM((_T, _DH), f32),
            pltpu.VMEM((_T, _DH), f32),
            pltpu.VMEM((_T, _DH), f32),
            pltpu.VMEM((_T, _DH), f32),
            pltpu.VMEM((_T, _H + 1), bf16),
            pltpu.VMEM((_T, 16), f32),
            pltpu.VMEM((_BB, _H), f32),
            pltpu.VMEM((_BB, _H), f32),
        ],
        compiler_params=pltpu.CompilerParams(
            dimension_semantics=("parallel",),
        ),
    )(*operands)

    return jnp.sum(partial[:, 0, 0]) * (1.0 / _B)
